# Initial kernel scaffold; baseline (speedup 1.0000x reference)
import jax
import jax.numpy as jnp
from jax.experimental import pallas as pl


def kernel(node_fts, edge_fts, edges, Wh, We, a_src, a_dst, a_edge):
    raise NotImplementedError("write your pallas kernel here")


# speedup vs baseline: 12.9574x; 12.9574x over previous
import functools
import jax
import jax.numpy as jnp
from jax import lax
from jax.experimental import pallas as pl
from jax.experimental.pallas import tpu as pltpu
from jax.experimental.pallas import tpu_sc as plsc

N = 10000
E = 320000
DIN = 128
DE = 16
DO = 64
H = 4
ALPHA = 0.2

NP_ = 10240
NC = 2
NS = 16
NW = NC * NS
EW = E // NW
CH = 80
NCH = EW // CH
STRIPE = NP_ // NS

BN = 2048
BE = 12800
BND = 2000


def _prep_nodes_body(node_ref, wh_ref, asrc_ref, adst_ref,
                     h_ref, ssrc_ref, sdst_ref, smax_ref):
    nb = pl.program_id(1)
    x = node_ref[...]
    hb = jnp.dot(x, wh_ref[0], preferred_element_type=jnp.float32)
    h_ref[0] = hb
    ss = jnp.dot(hb, asrc_ref[0, 0], preferred_element_type=jnp.float32)
    sd = jnp.dot(hb, adst_ref[0, 0], preferred_element_type=jnp.float32)
    ssrc_ref[0, 0] = ss
    sdst_ref[0, 0] = sd
    mx = jnp.max(ss)

    @pl.when(nb == 0)
    def _():
        smax_ref[0, 0] = jnp.full((16,), mx, jnp.float32)

    @pl.when(nb != 0)
    def _():
        smax_ref[0, 0] = jnp.maximum(smax_ref[0, 0], mx)


def _prep_nodes(node_p, Wh, a_src, a_dst):
    nblk = NP_ // BN
    return pl.pallas_call(
        _prep_nodes_body,
        grid=(H, nblk),
        in_specs=[
            pl.BlockSpec((BN, DIN), lambda i, nb: (nb, 0)),
            pl.BlockSpec((1, DIN, DO), lambda i, nb: (i, 0, 0)),
            pl.BlockSpec((1, 1, DO), lambda i, nb: (i, 0, 0)),
            pl.BlockSpec((1, 1, DO), lambda i, nb: (i, 0, 0)),
        ],
        out_specs=[
            pl.BlockSpec((1, BN, DO), lambda i, nb: (i, nb, 0)),
            pl.BlockSpec((1, 1, BN), lambda i, nb: (i, 0, nb)),
            pl.BlockSpec((1, 1, BN), lambda i, nb: (i, 0, nb)),
            pl.BlockSpec((1, 1, 16), lambda i, nb: (i, 0, 0)),
        ],
        out_shape=[
            jax.ShapeDtypeStruct((H, NP_, DO), jnp.float32),
            jax.ShapeDtypeStruct((H, 1, NP_), jnp.float32),
            jax.ShapeDtypeStruct((H, 1, NP_), jnp.float32),
            jax.ShapeDtypeStruct((H, 1, 16), jnp.float32),
        ],
    )(node_p, Wh, a_src.reshape(H, 1, DO), a_dst.reshape(H, 1, DO))


def _prep_edges_body(ef_ref, we_ref, ae_ref, t_ref, tmax_ref):
    eb = pl.program_id(1)
    v = jnp.dot(we_ref[0], ae_ref[0, 0], preferred_element_type=jnp.float32)
    tb = jnp.dot(ef_ref[...], v, preferred_element_type=jnp.float32)
    t_ref[0, 0] = tb
    mx = jnp.max(tb)

    @pl.when(eb == 0)
    def _():
        tmax_ref[0, 0] = jnp.full((16,), mx, jnp.float32)

    @pl.when(eb != 0)
    def _():
        tmax_ref[0, 0] = jnp.maximum(tmax_ref[0, 0], mx)


def _prep_edges(edge_fts, We, a_edge):
    return pl.pallas_call(
        _prep_edges_body,
        grid=(H, E // BE),
        in_specs=[
            pl.BlockSpec((BE, DE), lambda i, eb: (eb, 0)),
            pl.BlockSpec((1, DE, DO), lambda i, eb: (i, 0, 0)),
            pl.BlockSpec((1, 1, DO), lambda i, eb: (i, 0, 0)),
        ],
        out_specs=[
            pl.BlockSpec((1, 1, BE), lambda i, eb: (i, 0, eb)),
            pl.BlockSpec((1, 1, 16), lambda i, eb: (i, 0, 0)),
        ],
        out_shape=[
            jax.ShapeDtypeStruct((H, 1, E), jnp.float32),
            jax.ShapeDtypeStruct((H, 1, 16), jnp.float32),
        ],
    )(edge_fts, We, a_edge.reshape(H, 1, DO))


def _sc_pass1_body(ssrc_hbm, sdst_hbm, t_hbm, srcr_hbm, dstr_hbm,
                   smax_hbm, tmax_hbm,
                   p_hbm, den_hbm,
                   src2d, dst2d, t2d, p2d, stab, dtab, mv1, mv2,
                   dpriv, dsh, dbuf, abuf):
    c = lax.axis_index("c")
    s = lax.axis_index("s")
    wid = c * NS + s
    pltpu.sync_copy(srcr_hbm.at[wid], src2d)
    pltpu.sync_copy(dstr_hbm.at[wid], dst2d)
    zero16 = jnp.zeros((16,), jnp.float32)
    for i in range(H):
        pltpu.sync_copy(ssrc_hbm.at[i, 0], stab)
        pltpu.sync_copy(sdst_hbm.at[i, 0], dtab)
        pltpu.sync_copy(t_hbm.at[i, wid], t2d)
        pltpu.sync_copy(smax_hbm.at[i, 0], mv1)
        pltpu.sync_copy(tmax_hbm.at[i, 0], mv2)
        mvv = mv1[...] + mv2[...]

        def _zpriv(k, _):
            dpriv[pl.ds(k * 16, 16)] = zero16
            return 0
        lax.fori_loop(0, NP_ // 16, _zpriv, 0)

        def _chunk(j, _):
            for q in range(CH // 16):
                sl = pl.ds(q * 16, 16)
                si = src2d[j, sl]
                di = dst2d[j, sl]
                a = plsc.load_gather(stab, [si])
                b = plsc.load_gather(dtab, [di])
                e = a + b + t2d[j, sl]
                z = jnp.where(e >= 0, e, e * ALPHA)
                u = b + mvv
                cv = jnp.where(u >= 0, u, u * ALPHA)
                p16 = jnp.exp(z - cv)
                p2d[j, sl] = p16
                plsc.addupdate_scatter(dpriv, [di], p16)
            return 0
        lax.fori_loop(0, NCH, _chunk, 0)

        pltpu.sync_copy(p2d, p_hbm.at[i, wid])
        pltpu.sync_copy(dpriv, dsh.at[s])
        plsc.subcore_barrier()

        def _zab(k, _):
            abuf[pl.ds(k * 16, 16)] = zero16
            return 0
        lax.fori_loop(0, STRIPE // 16, _zab, 0)
        for m in range(NS):
            pltpu.sync_copy(dsh.at[m, pl.ds(s * STRIPE, STRIPE)], dbuf)

            def _acc(k, _):
                sl = pl.ds(k * 16, 16)
                abuf[sl] = abuf[sl] + dbuf[sl]
                return 0
            lax.fori_loop(0, STRIPE // 16, _acc, 0)
        pltpu.sync_copy(abuf, den_hbm.at[c, i, pl.ds(s * STRIPE, STRIPE)])
        plsc.subcore_barrier()


def _sc_pass1(s_src, s_dst, t, srcr, dstr, smax, tmax):
    mesh = plsc.VectorSubcoreMesh(core_axis_name="c", subcore_axis_name="s")
    return pl.kernel(
        _sc_pass1_body,
        out_type=[
            jax.ShapeDtypeStruct((H, NW, NCH, CH), jnp.float32),
            jax.ShapeDtypeStruct((NC, H, NP_), jnp.float32),
        ],
        mesh=mesh,
        compiler_params=pltpu.CompilerParams(needs_layout_passes=False),
        scratch_types=[
            pltpu.VMEM((NCH, CH), jnp.int32),
            pltpu.VMEM((NCH, CH), jnp.int32),
            pltpu.VMEM((NCH, CH), jnp.float32),
            pltpu.VMEM((NCH, CH), jnp.float32),
            pltpu.VMEM((NP_,), jnp.float32),
            pltpu.VMEM((NP_,), jnp.float32),
            pltpu.VMEM((16,), jnp.float32),
            pltpu.VMEM((16,), jnp.float32),
            pltpu.VMEM((NP_,), jnp.float32),
            pltpu.VMEM_SHARED((NS, NP_), jnp.float32),
            pltpu.VMEM((STRIPE,), jnp.float32),
            pltpu.VMEM((STRIPE,), jnp.float32),
        ],
    )(s_src, s_dst, t, srcr, dstr, smax, tmax)


def _rdenom_body(den_ref, out_ref):
    d = den_ref[0] + den_ref[1]
    out_ref[...] = 1.0 / (d + 1e-16)


def _rdenom(den):
    return pl.pallas_call(
        _rdenom_body,
        grid=(H,),
        in_specs=[pl.BlockSpec((NC, NP_), lambda i: (0, i))],
        out_specs=pl.BlockSpec((NP_,), lambda i: (i,)),
        out_shape=jax.ShapeDtypeStruct((H * NP_,), jnp.float32),
    )(den.reshape(NC, H * NP_))


def _sc_pass2_body(p_hbm, rden_hbm, srcr_hbm, dstr_hbm, h2_hbm, ef_hbm,
                   o1_hbm, g_hbm, s1_hbm, s2_hbm,
                   src2d, dst2d, p2d, rtab, idxv, attv, hbuf, efbuf,
                   o1sh, gsh, zo1, zg, sbuf, sem):
    c = lax.axis_index("c")
    s = lax.axis_index("s")
    wid = c * NS + s
    base = wid * EW
    pltpu.sync_copy(srcr_hbm.at[wid], src2d)
    pltpu.sync_copy(dstr_hbm.at[wid], dst2d)
    zero16 = jnp.zeros((16,), jnp.float32)
    for i in range(H):
        pltpu.sync_copy(p_hbm.at[i, wid], p2d)
        pltpu.sync_copy(rden_hbm.at[pl.ds(i * NP_, NP_)], rtab)

        def _zrow(k, _):
            for m in range(DO // 16):
                zo1[k, pl.ds(m * 16, 16)] = zero16
            zg[k, pl.ds(0, 16)] = zero16
            return 0
        lax.fori_loop(0, CH, _zrow, 0)

        def _zpub(k, _):
            pltpu.sync_copy(zo1, o1sh.at[pl.ds(s * STRIPE + k * CH, CH)])
            pltpu.sync_copy(zg, gsh.at[pl.ds(s * STRIPE + k * CH, CH)])
            return 0
        lax.fori_loop(0, STRIPE // CH, _zpub, 0)
        plsc.subcore_barrier()

        ioff = i * NP_

        def _chunk(j, carry):
            s1v, s2v = carry
            for q in range(CH // 16):
                sl = pl.ds(q * 16, 16)
                idxv[sl] = src2d[j, sl] + ioff
            pltpu.async_copy(h2_hbm.at[idxv], hbuf, sem).wait()
            pltpu.sync_copy(ef_hbm.at[pl.ds(base + j * CH, CH)], efbuf)
            for q in range(CH // 16):
                sl = pl.ds(q * 16, 16)
                di = dst2d[j, sl]
                r16 = plsc.load_gather(rtab, [di])
                a16 = p2d[j, sl] * r16
                s1v = s1v + a16
                s2v = s2v + a16 * a16
                for r in range(16):
                    rr = q * 16 + r
                    av = a16[r]
                    for m in range(DO // 16):
                        sl2 = pl.ds(m * 16, 16)
                        hbuf[rr, sl2] = hbuf[rr, sl2] * av
                    efbuf[rr, pl.ds(0, 16)] = efbuf[rr, pl.ds(0, 16)] * av
            pltpu.sync_copy(hbuf, o1sh.at[dst2d.at[j]], add=True)
            pltpu.sync_copy(efbuf, gsh.at[dst2d.at[j]], add=True)
            return (s1v, s2v)

        s1v, s2v = lax.fori_loop(0, NCH, _chunk, (zero16, zero16))
        plsc.subcore_barrier()

        def _expo(k, _):
            off = s * STRIPE + k * CH
            pltpu.sync_copy(o1sh.at[pl.ds(off, CH)], zo1)
            pltpu.sync_copy(zo1, o1_hbm.at[c, i, pl.ds(off, CH)])
            pltpu.sync_copy(gsh.at[pl.ds(off, CH)], zg)
            pltpu.sync_copy(zg, g_hbm.at[c, i, pl.ds(off, CH)])
            return 0
        lax.fori_loop(0, STRIPE // CH, _expo, 0)
        sbuf[pl.ds(0, 16)] = s1v
        pltpu.sync_copy(sbuf, s1_hbm.at[c, i, s])
        sbuf[pl.ds(0, 16)] = s2v
        pltpu.sync_copy(sbuf, s2_hbm.at[c, i, s])
        plsc.subcore_barrier()


def _sc_pass2(p, rden, srcr, dstr, h2, edge_fts):
    mesh = plsc.VectorSubcoreMesh(core_axis_name="c", subcore_axis_name="s")
    return pl.kernel(
        _sc_pass2_body,
        out_type=[
            jax.ShapeDtypeStruct((NC, H, NP_, DO), jnp.float32),
            jax.ShapeDtypeStruct((NC, H, NP_, DE), jnp.float32),
            jax.ShapeDtypeStruct((NC, H, NS, 16), jnp.float32),
            jax.ShapeDtypeStruct((NC, H, NS, 16), jnp.float32),
        ],
        mesh=mesh,
        compiler_params=pltpu.CompilerParams(
            needs_layout_passes=False, use_tc_tiling_on_sc=False),
        scratch_types=[
            pltpu.VMEM((NCH, CH), jnp.int32),
            pltpu.VMEM((NCH, CH), jnp.int32),
            pltpu.VMEM((NCH, CH), jnp.float32),
            pltpu.VMEM((NP_,), jnp.float32),
            pltpu.VMEM((CH,), jnp.int32),
            pltpu.VMEM((CH,), jnp.float32),
            pltpu.VMEM((CH, DO), jnp.float32),
            pltpu.VMEM((CH, DE), jnp.float32),
            pltpu.VMEM_SHARED((NP_, DO), jnp.float32),
            pltpu.VMEM_SHARED((NP_, DE), jnp.float32),
            pltpu.VMEM((CH, DO), jnp.float32),
            pltpu.VMEM((CH, DE), jnp.float32),
            pltpu.VMEM((16,), jnp.float32),
            pltpu.SemaphoreType.DMA,
        ],
    )(p, rden, srcr, dstr, h2, edge_fts)


def _final_body(o0_ref, o1_ref, g0_ref, g1_ref, we_ref, s1_ref, s2_ref,
                out_ref):
    s1 = jnp.sum(s1_ref[...], axis=(0, 2, 3))
    s2 = jnp.sum(s2_ref[...], axis=(0, 2, 3))
    fe = jnp.float32(E)
    var = s2 / fe - (s1 / fe) ** 2
    w = jnp.exp(var)
    w = w / jnp.sum(w)
    for i in range(H):
        g = g0_ref[i] + g1_ref[i]
        acc = o0_ref[i] + o1_ref[i] + jnp.dot(
            g, we_ref[i], preferred_element_type=jnp.float32)
        out_ref[:, i * DO:(i + 1) * DO] = acc * w[i]


def _final(o0, o1, g0, g1, We, s1, s2):
    nblk = N // BND
    return pl.pallas_call(
        _final_body,
        grid=(nblk,),
        in_specs=[
            pl.BlockSpec((H, BND, DO), lambda nb: (0, nb, 0)),
            pl.BlockSpec((H, BND, DO), lambda nb: (0, nb, 0)),
            pl.BlockSpec((H, BND, DE), lambda nb: (0, nb, 0)),
            pl.BlockSpec((H, BND, DE), lambda nb: (0, nb, 0)),
            pl.BlockSpec((H, DE, DO), lambda nb: (0, 0, 0)),
            pl.BlockSpec((NC, H, NS, 16), lambda nb: (0, 0, 0, 0)),
            pl.BlockSpec((NC, H, NS, 16), lambda nb: (0, 0, 0, 0)),
        ],
        out_specs=pl.BlockSpec((BND, H * DO), lambda nb: (nb, 0)),
        out_shape=jax.ShapeDtypeStruct((N, H * DO), jnp.float32),
    )(o0, o1, g0, g1, We, s1, s2)


@jax.jit
def kernel(node_fts, edge_fts, edges, Wh, We, a_src, a_dst, a_edge):
    node_p = jnp.pad(node_fts, ((0, NP_ - N), (0, 0)))
    edges32 = edges.astype(jnp.int32)
    srcr = edges32[:, 0].reshape(NW, NCH, CH)
    dstr = edges32[:, 1].reshape(NW, NCH, CH)

    h, s_src, s_dst, smax = _prep_nodes(node_p, Wh, a_src, a_dst)
    t, tmax = _prep_edges(edge_fts, We, a_edge)
    t_r = t.reshape(H, NW, NCH, CH)

    p, den = _sc_pass1(s_src, s_dst, t_r, srcr, dstr, smax, tmax)
    rden = _rdenom(den)

    h2 = h.reshape(H * NP_, DO)
    o1p, gp, s1, s2 = _sc_pass2(p, rden, srcr, dstr, h2, edge_fts)

    return _final(o1p[0], o1p[1], gp[0], gp[1], We, s1, s2)


# speedup vs baseline: 18.9121x; 1.4596x over previous
import functools
import jax
import jax.numpy as jnp
from jax import lax
from jax.experimental import pallas as pl
from jax.experimental.pallas import tpu as pltpu
from jax.experimental.pallas import tpu_sc as plsc

N = 10000
E = 320000
DIN = 128
DE = 16
DO = 64
H = 4
ALPHA = 0.2

NP_ = 10240
NC = 2
NS = 16
NW = NC * NS
EW = E // NW
CH = 80
NCH = EW // CH
STRIPE = NP_ // NS

BN = 2048
BE = 12800
BND = 2000


def _prep_nodes_body(node_ref, wh_ref, h_ref):
    x = node_ref[...]
    h_ref[0] = jnp.dot(x, wh_ref[0], preferred_element_type=jnp.float32)


def _prep_nodes(node_p, Wh):
    nblk = NP_ // BN
    return pl.pallas_call(
        _prep_nodes_body,
        grid=(H, nblk),
        in_specs=[
            pl.BlockSpec((BN, DIN), lambda i, nb: (nb, 0)),
            pl.BlockSpec((1, DIN, DO), lambda i, nb: (i, 0, 0)),
        ],
        out_specs=pl.BlockSpec((1, BN, DO), lambda i, nb: (i, nb, 0)),
        out_shape=jax.ShapeDtypeStruct((H, NP_, DO), jnp.float32),
    )(node_p, Wh)


def _prep_s_body(xt_ref, wht_ref, asrc_ref, adst_ref, s8_ref, smax_ref):
    nb = pl.program_id(0)
    rows = [jnp.dot(asrc_ref[i], wht_ref[i], preferred_element_type=jnp.float32)
            for i in range(H)]
    rows += [jnp.dot(adst_ref[i], wht_ref[i], preferred_element_type=jnp.float32)
             for i in range(H)]
    wst = jnp.concatenate(rows, axis=0)
    sblk = jnp.dot(wst, xt_ref[...], preferred_element_type=jnp.float32)
    s8_ref[...] = sblk
    mx = jnp.max(sblk, axis=1)
    bc = jnp.broadcast_to(mx[:, None], (2 * H, 16))

    @pl.when(nb == 0)
    def _():
        smax_ref[:, 0, :] = bc

    @pl.when(nb != 0)
    def _():
        smax_ref[:, 0, :] = jnp.maximum(smax_ref[:, 0, :], bc)


def _prep_s(node_pT, WhT, a_src, a_dst):
    nblk = NP_ // BN
    return pl.pallas_call(
        _prep_s_body,
        grid=(nblk,),
        in_specs=[
            pl.BlockSpec((DIN, BN), lambda nb: (0, nb)),
            pl.BlockSpec((H, DO, DIN), lambda nb: (0, 0, 0)),
            pl.BlockSpec((H, 1, DO), lambda nb: (0, 0, 0)),
            pl.BlockSpec((H, 1, DO), lambda nb: (0, 0, 0)),
        ],
        out_specs=[
            pl.BlockSpec((2 * H, BN), lambda nb: (0, nb)),
            pl.BlockSpec((2 * H, 1, 16), lambda nb: (0, 0, 0)),
        ],
        out_shape=[
            jax.ShapeDtypeStruct((2 * H, NP_), jnp.float32),
            jax.ShapeDtypeStruct((2 * H, 1, 16), jnp.float32),
        ],
    )(node_pT, WhT, a_src.reshape(H, 1, DO), a_dst.reshape(H, 1, DO))


def _prep_edges_body(eft_ref, wet_ref, ae_ref, t_ref, tmax_ref):
    eb = pl.program_id(0)
    rows = [jnp.dot(ae_ref[i], wet_ref[i], preferred_element_type=jnp.float32)
            for i in range(H)]
    v4 = jnp.concatenate(rows, axis=0)
    tblk = jnp.dot(v4, eft_ref[...], preferred_element_type=jnp.float32)
    t_ref[...] = tblk
    mx = jnp.max(tblk, axis=1)
    bc = jnp.broadcast_to(mx[:, None], (H, 16))

    @pl.when(eb == 0)
    def _():
        tmax_ref[:, 0, :] = bc

    @pl.when(eb != 0)
    def _():
        tmax_ref[:, 0, :] = jnp.maximum(tmax_ref[:, 0, :], bc)


def _prep_edges(edge_ftsT, WeT, a_edge):
    return pl.pallas_call(
        _prep_edges_body,
        grid=(E // BE,),
        in_specs=[
            pl.BlockSpec((DE, BE), lambda eb: (0, eb)),
            pl.BlockSpec((H, DO, DE), lambda eb: (0, 0, 0)),
            pl.BlockSpec((H, 1, DO), lambda eb: (0, 0, 0)),
        ],
        out_specs=[
            pl.BlockSpec((H, BE), lambda eb: (0, eb)),
            pl.BlockSpec((H, 1, 16), lambda eb: (0, 0, 0)),
        ],
        out_shape=[
            jax.ShapeDtypeStruct((H, E), jnp.float32),
            jax.ShapeDtypeStruct((H, 1, 16), jnp.float32),
        ],
    )(edge_ftsT, WeT, a_edge.reshape(H, 1, DO))


def _sc_pass1_body(s8_hbm, t_hbm, srcr_hbm, dstr_hbm,
                   smax_hbm, tmax_hbm,
                   p_hbm, den_hbm,
                   src2d, dst2d, t2d, p2d, stab, dtab, mv1, mv2,
                   dpriv, dsh, dbuf, abuf):
    c = lax.axis_index("c")
    s = lax.axis_index("s")
    wid = c * NS + s
    pltpu.sync_copy(srcr_hbm.at[wid], src2d)
    pltpu.sync_copy(dstr_hbm.at[wid], dst2d)
    zero16 = jnp.zeros((16,), jnp.float32)
    for i in range(H):
        pltpu.sync_copy(s8_hbm.at[i], stab)
        pltpu.sync_copy(s8_hbm.at[H + i], dtab)
        pltpu.sync_copy(t_hbm.at[i, wid], t2d)
        pltpu.sync_copy(smax_hbm.at[i, 0], mv1)
        pltpu.sync_copy(tmax_hbm.at[i, 0], mv2)
        mvv = mv1[...] + mv2[...]

        def _zpriv(k, _):
            dpriv[pl.ds(k * 16, 16)] = zero16
            return 0
        lax.fori_loop(0, NP_ // 16, _zpriv, 0)

        def _chunk(j, _):
            for q in range(CH // 16):
                sl = pl.ds(q * 16, 16)
                si = src2d[j, sl]
                di = dst2d[j, sl]
                a = plsc.load_gather(stab, [si])
                b = plsc.load_gather(dtab, [di])
                e = a + b + t2d[j, sl]
                z = jnp.where(e >= 0, e, e * ALPHA)
                u = b + mvv
                cv = jnp.where(u >= 0, u, u * ALPHA)
                p16 = jnp.exp(z - cv)
                p2d[j, sl] = p16
                plsc.addupdate_scatter(dpriv, [di], p16)
            return 0
        lax.fori_loop(0, NCH, _chunk, 0)

        pltpu.sync_copy(p2d, p_hbm.at[i, wid])
        pltpu.sync_copy(dpriv, dsh.at[s])
        plsc.subcore_barrier()

        def _zab(k, _):
            abuf[pl.ds(k * 16, 16)] = zero16
            return 0
        lax.fori_loop(0, STRIPE // 16, _zab, 0)
        for m in range(NS):
            pltpu.sync_copy(dsh.at[m, pl.ds(s * STRIPE, STRIPE)], dbuf)

            def _acc(k, _):
                sl = pl.ds(k * 16, 16)
                abuf[sl] = abuf[sl] + dbuf[sl]
                return 0
            lax.fori_loop(0, STRIPE // 16, _acc, 0)
        pltpu.sync_copy(abuf, den_hbm.at[c, i, pl.ds(s * STRIPE, STRIPE)])
        plsc.subcore_barrier()


def _sc_pass1(s8, t, srcr, dstr, smax, tmax):
    mesh = plsc.VectorSubcoreMesh(core_axis_name="c", subcore_axis_name="s")
    return pl.kernel(
        _sc_pass1_body,
        out_type=[
            jax.ShapeDtypeStruct((H, NW, NCH, CH), jnp.float32),
            jax.ShapeDtypeStruct((NC, H, NP_), jnp.float32),
        ],
        mesh=mesh,
        compiler_params=pltpu.CompilerParams(needs_layout_passes=False),
        scratch_types=[
            pltpu.VMEM((NCH, CH), jnp.int32),
            pltpu.VMEM((NCH, CH), jnp.int32),
            pltpu.VMEM((NCH, CH), jnp.float32),
            pltpu.VMEM((NCH, CH), jnp.float32),
            pltpu.VMEM((NP_,), jnp.float32),
            pltpu.VMEM((NP_,), jnp.float32),
            pltpu.VMEM((16,), jnp.float32),
            pltpu.VMEM((16,), jnp.float32),
            pltpu.VMEM((NP_,), jnp.float32),
            pltpu.VMEM_SHARED((NS, NP_), jnp.float32),
            pltpu.VMEM((STRIPE,), jnp.float32),
            pltpu.VMEM((STRIPE,), jnp.float32),
        ],
    )(s8, t, srcr, dstr, smax, tmax)


def _rdenom_body(den_ref, out_ref):
    d = den_ref[0] + den_ref[1]
    out_ref[...] = 1.0 / (d + 1e-16)


def _rdenom(den):
    return pl.pallas_call(
        _rdenom_body,
        grid=(H,),
        in_specs=[pl.BlockSpec((NC, NP_), lambda i: (0, i))],
        out_specs=pl.BlockSpec((NP_,), lambda i: (i,)),
        out_shape=jax.ShapeDtypeStruct((H * NP_,), jnp.float32),
    )(den.reshape(NC, H * NP_))


def _sc_pass2_body(p_hbm, rden_hbm, srcr_hbm, dstr_hbm, h2_hbm, ef_hbm,
                   o1_hbm, g_hbm, s1_hbm, s2_hbm,
                   src2d, dst2d, p2d, rtab, idxv, attv, hbuf, efbuf,
                   o1sh, gsh, zo1, zg, sbuf, sem):
    c = lax.axis_index("c")
    s = lax.axis_index("s")
    wid = c * NS + s
    base = wid * EW
    pltpu.sync_copy(srcr_hbm.at[wid], src2d)
    pltpu.sync_copy(dstr_hbm.at[wid], dst2d)
    zero16 = jnp.zeros((16,), jnp.float32)
    for i in range(H):
        pltpu.sync_copy(p_hbm.at[i, wid], p2d)
        pltpu.sync_copy(rden_hbm.at[pl.ds(i * NP_, NP_)], rtab)

        def _zrow(k, _):
            for m in range(DO // 16):
                zo1[k, pl.ds(m * 16, 16)] = zero16
            zg[k, pl.ds(0, 16)] = zero16
            return 0
        lax.fori_loop(0, CH, _zrow, 0)

        def _zpub(k, _):
            pltpu.sync_copy(zo1, o1sh.at[pl.ds(s * STRIPE + k * CH, CH)])
            pltpu.sync_copy(zg, gsh.at[pl.ds(s * STRIPE + k * CH, CH)])
            return 0
        lax.fori_loop(0, STRIPE // CH, _zpub, 0)
        plsc.subcore_barrier()

        ioff = i * NP_

        def _chunk(j, carry):
            s1v, s2v = carry
            for q in range(CH // 16):
                sl = pl.ds(q * 16, 16)
                idxv[sl] = src2d[j, sl] + ioff
            pltpu.async_copy(h2_hbm.at[idxv], hbuf, sem).wait()
            pltpu.sync_copy(ef_hbm.at[pl.ds(base + j * CH, CH)], efbuf)
            for q in range(CH // 16):
                sl = pl.ds(q * 16, 16)
                di = dst2d[j, sl]
                r16 = plsc.load_gather(rtab, [di])
                a16 = p2d[j, sl] * r16
                s1v = s1v + a16
                s2v = s2v + a16 * a16
                for r in range(16):
                    rr = q * 16 + r
                    av = a16[r]
                    for m in range(DO // 16):
                        sl2 = pl.ds(m * 16, 16)
                        hbuf[rr, sl2] = hbuf[rr, sl2] * av
                    efbuf[rr, pl.ds(0, 16)] = efbuf[rr, pl.ds(0, 16)] * av
            pltpu.sync_copy(hbuf, o1sh.at[dst2d.at[j]], add=True)
            pltpu.sync_copy(efbuf, gsh.at[dst2d.at[j]], add=True)
            return (s1v, s2v)

        s1v, s2v = lax.fori_loop(0, NCH, _chunk, (zero16, zero16))
        plsc.subcore_barrier()

        def _expo(k, _):
            off = s * STRIPE + k * CH
            pltpu.sync_copy(o1sh.at[pl.ds(off, CH)], zo1)
            pltpu.sync_copy(zo1, o1_hbm.at[c, i, pl.ds(off, CH)])
            pltpu.sync_copy(gsh.at[pl.ds(off, CH)], zg)
            pltpu.sync_copy(zg, g_hbm.at[c, i, pl.ds(off, CH)])
            return 0
        lax.fori_loop(0, STRIPE // CH, _expo, 0)
        sbuf[pl.ds(0, 16)] = s1v
        pltpu.sync_copy(sbuf, s1_hbm.at[c, i, s])
        sbuf[pl.ds(0, 16)] = s2v
        pltpu.sync_copy(sbuf, s2_hbm.at[c, i, s])
        plsc.subcore_barrier()


def _sc_pass2(p, rden, srcr, dstr, h2, edge_fts):
    mesh = plsc.VectorSubcoreMesh(core_axis_name="c", subcore_axis_name="s")
    return pl.kernel(
        _sc_pass2_body,
        out_type=[
            jax.ShapeDtypeStruct((NC, H, NP_, DO), jnp.float32),
            jax.ShapeDtypeStruct((NC, H, NP_, DE), jnp.float32),
            jax.ShapeDtypeStruct((NC, H, NS, 16), jnp.float32),
            jax.ShapeDtypeStruct((NC, H, NS, 16), jnp.float32),
        ],
        mesh=mesh,
        compiler_params=pltpu.CompilerParams(
            needs_layout_passes=False, use_tc_tiling_on_sc=False),
        scratch_types=[
            pltpu.VMEM((NCH, CH), jnp.int32),
            pltpu.VMEM((NCH, CH), jnp.int32),
            pltpu.VMEM((NCH, CH), jnp.float32),
            pltpu.VMEM((NP_,), jnp.float32),
            pltpu.VMEM((CH,), jnp.int32),
            pltpu.VMEM((CH,), jnp.float32),
            pltpu.VMEM((CH, DO), jnp.float32),
            pltpu.VMEM((CH, DE), jnp.float32),
            pltpu.VMEM_SHARED((NP_, DO), jnp.float32),
            pltpu.VMEM_SHARED((NP_, DE), jnp.float32),
            pltpu.VMEM((CH, DO), jnp.float32),
            pltpu.VMEM((CH, DE), jnp.float32),
            pltpu.VMEM((16,), jnp.float32),
            pltpu.SemaphoreType.DMA,
        ],
    )(p, rden, srcr, dstr, h2, edge_fts)


def _final_body(o0_ref, o1_ref, g0_ref, g1_ref, we_ref, s1_ref, s2_ref,
                out_ref):
    s1 = jnp.sum(s1_ref[...], axis=(0, 2, 3))
    s2 = jnp.sum(s2_ref[...], axis=(0, 2, 3))
    fe = jnp.float32(E)
    var = s2 / fe - (s1 / fe) ** 2
    w = jnp.exp(var)
    w = w / jnp.sum(w)
    for i in range(H):
        g = g0_ref[i] + g1_ref[i]
        acc = o0_ref[i] + o1_ref[i] + jnp.dot(
            g, we_ref[i], preferred_element_type=jnp.float32)
        out_ref[:, i * DO:(i + 1) * DO] = acc * w[i]


def _final(o0, o1, g0, g1, We, s1, s2):
    nblk = N // BND
    return pl.pallas_call(
        _final_body,
        grid=(nblk,),
        in_specs=[
            pl.BlockSpec((H, BND, DO), lambda nb: (0, nb, 0)),
            pl.BlockSpec((H, BND, DO), lambda nb: (0, nb, 0)),
            pl.BlockSpec((H, BND, DE), lambda nb: (0, nb, 0)),
            pl.BlockSpec((H, BND, DE), lambda nb: (0, nb, 0)),
            pl.BlockSpec((H, DE, DO), lambda nb: (0, 0, 0)),
            pl.BlockSpec((NC, H, NS, 16), lambda nb: (0, 0, 0, 0)),
            pl.BlockSpec((NC, H, NS, 16), lambda nb: (0, 0, 0, 0)),
        ],
        out_specs=pl.BlockSpec((BND, H * DO), lambda nb: (nb, 0)),
        out_shape=jax.ShapeDtypeStruct((N, H * DO), jnp.float32),
    )(o0, o1, g0, g1, We, s1, s2)


@jax.jit
def kernel(node_fts, edge_fts, edges, Wh, We, a_src, a_dst, a_edge):
    node_p = jnp.pad(node_fts, ((0, NP_ - N), (0, 0)))
    edges32 = edges.astype(jnp.int32)
    srcr = edges32[:, 0].reshape(NW, NCH, CH)
    dstr = edges32[:, 1].reshape(NW, NCH, CH)

    h = _prep_nodes(node_p, Wh)
    s8, smax8 = _prep_s(node_p.T, Wh.transpose(0, 2, 1), a_src, a_dst)
    t, tmax = _prep_edges(edge_fts.T, We.transpose(0, 2, 1), a_edge)
    t_r = t.reshape(H, NW, NCH, CH)

    p, den = _sc_pass1(s8, t_r, srcr, dstr, smax8, tmax)
    rden = _rdenom(den)

    h2 = h.reshape(H * NP_, DO)
    o1p, gp, s1, s2 = _sc_pass2(p, rden, srcr, dstr, h2, edge_fts)

    return _final(o1p[0], o1p[1], gp[0], gp[1], We, s1, s2)


# speedup vs baseline: 25.2068x; 1.3328x over previous
import functools
import jax
import jax.numpy as jnp
from jax import lax
from jax.experimental import pallas as pl
from jax.experimental.pallas import tpu as pltpu
from jax.experimental.pallas import tpu_sc as plsc

N = 10000
E = 320000
DIN = 128
DE = 16
DO = 64
H = 4
ALPHA = 0.2

NP_ = 10240
NC = 2
NS = 16
NW = NC * NS
EW = E // NW
CH = 80
NCH = EW // CH
STRIPE = NP_ // NS

BN = 2048
BE = 12800
BND = 2000


def _prep_nodes_body(node_ref, wh_ref, h_ref):
    x = node_ref[...]
    h_ref[0] = jnp.dot(x, wh_ref[0], preferred_element_type=jnp.float32)


def _prep_nodes(node_p, Wh):
    nblk = NP_ // BN
    return pl.pallas_call(
        _prep_nodes_body,
        grid=(H, nblk),
        in_specs=[
            pl.BlockSpec((BN, DIN), lambda i, nb: (nb, 0)),
            pl.BlockSpec((1, DIN, DO), lambda i, nb: (i, 0, 0)),
        ],
        out_specs=pl.BlockSpec((1, BN, DO), lambda i, nb: (i, nb, 0)),
        out_shape=jax.ShapeDtypeStruct((H, NP_, DO), jnp.float32),
    )(node_p, Wh)


def _prep_s_body(xt_ref, wht_ref, asrc_ref, adst_ref, s8_ref, smax_ref):
    nb = pl.program_id(0)
    rows = [jnp.dot(asrc_ref[i], wht_ref[i], preferred_element_type=jnp.float32)
            for i in range(H)]
    rows += [jnp.dot(adst_ref[i], wht_ref[i], preferred_element_type=jnp.float32)
             for i in range(H)]
    wst = jnp.concatenate(rows, axis=0)
    sblk = jnp.dot(wst, xt_ref[...], preferred_element_type=jnp.float32)
    s8_ref[...] = sblk
    mx = jnp.max(sblk, axis=1)
    bc = jnp.broadcast_to(mx[:, None], (2 * H, 16))

    @pl.when(nb == 0)
    def _():
        smax_ref[:, 0, :] = bc

    @pl.when(nb != 0)
    def _():
        smax_ref[:, 0, :] = jnp.maximum(smax_ref[:, 0, :], bc)


def _prep_s(node_pT, WhT, a_src, a_dst):
    nblk = NP_ // BN
    return pl.pallas_call(
        _prep_s_body,
        grid=(nblk,),
        in_specs=[
            pl.BlockSpec((DIN, BN), lambda nb: (0, nb)),
            pl.BlockSpec((H, DO, DIN), lambda nb: (0, 0, 0)),
            pl.BlockSpec((H, 1, DO), lambda nb: (0, 0, 0)),
            pl.BlockSpec((H, 1, DO), lambda nb: (0, 0, 0)),
        ],
        out_specs=[
            pl.BlockSpec((2 * H, BN), lambda nb: (0, nb)),
            pl.BlockSpec((2 * H, 1, 16), lambda nb: (0, 0, 0)),
        ],
        out_shape=[
            jax.ShapeDtypeStruct((2 * H, NP_), jnp.float32),
            jax.ShapeDtypeStruct((2 * H, 1, 16), jnp.float32),
        ],
    )(node_pT, WhT, a_src.reshape(H, 1, DO), a_dst.reshape(H, 1, DO))


def _prep_edges_body(eft_ref, wet_ref, ae_ref, t_ref, tmax_ref):
    eb = pl.program_id(0)
    rows = [jnp.dot(ae_ref[i], wet_ref[i], preferred_element_type=jnp.float32)
            for i in range(H)]
    v4 = jnp.concatenate(rows, axis=0)
    tblk = jnp.dot(v4, eft_ref[...], preferred_element_type=jnp.float32)
    t_ref[...] = tblk
    mx = jnp.max(tblk, axis=1)
    bc = jnp.broadcast_to(mx[:, None], (H, 16))

    @pl.when(eb == 0)
    def _():
        tmax_ref[:, 0, :] = bc

    @pl.when(eb != 0)
    def _():
        tmax_ref[:, 0, :] = jnp.maximum(tmax_ref[:, 0, :], bc)


def _prep_edges(edge_ftsT, WeT, a_edge):
    return pl.pallas_call(
        _prep_edges_body,
        grid=(E // BE,),
        in_specs=[
            pl.BlockSpec((DE, BE), lambda eb: (0, eb)),
            pl.BlockSpec((H, DO, DE), lambda eb: (0, 0, 0)),
            pl.BlockSpec((H, 1, DO), lambda eb: (0, 0, 0)),
        ],
        out_specs=[
            pl.BlockSpec((H, BE), lambda eb: (0, eb)),
            pl.BlockSpec((H, 1, 16), lambda eb: (0, 0, 0)),
        ],
        out_shape=[
            jax.ShapeDtypeStruct((H, E), jnp.float32),
            jax.ShapeDtypeStruct((H, 1, 16), jnp.float32),
        ],
    )(edge_ftsT, WeT, a_edge.reshape(H, 1, DO))


def _sc_pass1_body(s8_hbm, t_hbm, srcr_hbm, dstr_hbm,
                   smax_hbm, tmax_hbm,
                   p_hbm, den_hbm,
                   src2d, dst2d, t2d, p2d, stab, dtab, mv1, mv2,
                   dpriv, dsh, dbuf, abuf):
    c = lax.axis_index("c")
    s = lax.axis_index("s")
    wid = c * NS + s
    pltpu.sync_copy(srcr_hbm.at[wid], src2d)
    pltpu.sync_copy(dstr_hbm.at[wid], dst2d)
    zero16 = jnp.zeros((16,), jnp.float32)
    for i in range(H):
        pltpu.sync_copy(s8_hbm.at[i], stab)
        pltpu.sync_copy(s8_hbm.at[H + i], dtab)
        pltpu.sync_copy(t_hbm.at[i, wid], t2d)
        pltpu.sync_copy(smax_hbm.at[i, 0], mv1)
        pltpu.sync_copy(tmax_hbm.at[i, 0], mv2)
        mvv = mv1[...] + mv2[...]

        def _zpriv(k, _):
            dpriv[pl.ds(k * 16, 16)] = zero16
            return 0
        lax.fori_loop(0, NP_ // 16, _zpriv, 0)

        def _chunk(j, _):
            for q in range(CH // 16):
                sl = pl.ds(q * 16, 16)
                si = src2d[j, sl]
                di = dst2d[j, sl]
                a = plsc.load_gather(stab, [si])
                b = plsc.load_gather(dtab, [di])
                e = a + b + t2d[j, sl]
                z = jnp.where(e >= 0, e, e * ALPHA)
                u = b + mvv
                cv = jnp.where(u >= 0, u, u * ALPHA)
                p16 = jnp.exp(z - cv)
                p2d[j, sl] = p16
                plsc.addupdate_scatter(dpriv, [di], p16)
            return 0
        lax.fori_loop(0, NCH, _chunk, 0)

        pltpu.sync_copy(p2d, p_hbm.at[i, wid])
        pltpu.sync_copy(dpriv, dsh.at[s])
        plsc.subcore_barrier()

        def _zab(k, _):
            abuf[pl.ds(k * 16, 16)] = zero16
            return 0
        lax.fori_loop(0, STRIPE // 16, _zab, 0)
        for m in range(NS):
            pltpu.sync_copy(dsh.at[m, pl.ds(s * STRIPE, STRIPE)], dbuf)

            def _acc(k, _):
                sl = pl.ds(k * 16, 16)
                abuf[sl] = abuf[sl] + dbuf[sl]
                return 0
            lax.fori_loop(0, STRIPE // 16, _acc, 0)
        pltpu.sync_copy(abuf, den_hbm.at[c, i, pl.ds(s * STRIPE, STRIPE)])
        plsc.subcore_barrier()


def _sc_pass1(s8, t, srcr, dstr, smax, tmax):
    mesh = plsc.VectorSubcoreMesh(core_axis_name="c", subcore_axis_name="s")
    return pl.kernel(
        _sc_pass1_body,
        out_type=[
            jax.ShapeDtypeStruct((H, NW, NCH, CH), jnp.float32),
            jax.ShapeDtypeStruct((NC, H, NP_), jnp.float32),
        ],
        mesh=mesh,
        compiler_params=pltpu.CompilerParams(needs_layout_passes=False),
        scratch_types=[
            pltpu.VMEM((NCH, CH), jnp.int32),
            pltpu.VMEM((NCH, CH), jnp.int32),
            pltpu.VMEM((NCH, CH), jnp.float32),
            pltpu.VMEM((NCH, CH), jnp.float32),
            pltpu.VMEM((NP_,), jnp.float32),
            pltpu.VMEM((NP_,), jnp.float32),
            pltpu.VMEM((16,), jnp.float32),
            pltpu.VMEM((16,), jnp.float32),
            pltpu.VMEM((NP_,), jnp.float32),
            pltpu.VMEM_SHARED((NS, NP_), jnp.float32),
            pltpu.VMEM((STRIPE,), jnp.float32),
            pltpu.VMEM((STRIPE,), jnp.float32),
        ],
    )(s8, t, srcr, dstr, smax, tmax)


def _rdenom_body(den_ref, out_ref):
    d = den_ref[0] + den_ref[1]
    out_ref[...] = 1.0 / (d + 1e-16)


def _rdenom(den):
    return pl.pallas_call(
        _rdenom_body,
        grid=(H,),
        in_specs=[pl.BlockSpec((NC, NP_), lambda i: (0, i))],
        out_specs=pl.BlockSpec((NP_,), lambda i: (i,)),
        out_shape=jax.ShapeDtypeStruct((H * NP_,), jnp.float32),
    )(den.reshape(NC, H * NP_))


def _sc_pass2_body(p_hbm, rden_hbm, srcr_hbm, dstr_hbm, h2_hbm, ef_hbm,
                   o1_hbm, g_hbm, s1_hbm, s2_hbm,
                   src2d, dst2d, p2d, rtab, idxva, idxvb, hbufa, hbufb,
                   efbuf, o1sh, gsh, zo1, zg, sbuf, sema, semb):
    c = lax.axis_index("c")
    s = lax.axis_index("s")
    wid = c * NS + s
    base = wid * EW
    pltpu.sync_copy(srcr_hbm.at[wid], src2d)
    pltpu.sync_copy(dstr_hbm.at[wid], dst2d)
    zero16 = jnp.zeros((16,), jnp.float32)

    def _fill_idx(idxv, j, ioff):
        for q in range(CH // 16):
            sl = pl.ds(q * 16, 16)
            idxv[sl] = src2d[j, sl] + ioff

    def _process(j, hbuf, carry):
        s1v, s2v = carry
        pltpu.sync_copy(ef_hbm.at[pl.ds(base + j * CH, CH)], efbuf)
        for q in range(CH // 16):
            sl = pl.ds(q * 16, 16)
            di = dst2d[j, sl]
            r16 = plsc.load_gather(rtab, [di])
            a16 = p2d[j, sl] * r16
            s1v = s1v + a16
            s2v = s2v + a16 * a16
            for r in range(16):
                rr = q * 16 + r
                av = a16[r]
                for m in range(DO // 16):
                    sl2 = pl.ds(m * 16, 16)
                    hbuf[rr, sl2] = hbuf[rr, sl2] * av
                efbuf[rr, pl.ds(0, 16)] = efbuf[rr, pl.ds(0, 16)] * av
        pltpu.sync_copy(hbuf, o1sh.at[dst2d.at[j]], add=True)
        pltpu.sync_copy(efbuf, gsh.at[dst2d.at[j]], add=True)
        return (s1v, s2v)

    def _head(i, _):
        ioff = i * NP_
        pltpu.sync_copy(p_hbm.at[i, wid], p2d)
        pltpu.sync_copy(rden_hbm.at[pl.ds(i * NP_, NP_)], rtab)

        def _zrow(k, _):
            for m in range(DO // 16):
                zo1[k, pl.ds(m * 16, 16)] = zero16
            zg[k, pl.ds(0, 16)] = zero16
            return 0
        lax.fori_loop(0, CH, _zrow, 0)

        def _zpub(k, _):
            pltpu.sync_copy(zo1, o1sh.at[pl.ds(s * STRIPE + k * CH, CH)])
            pltpu.sync_copy(zg, gsh.at[pl.ds(s * STRIPE + k * CH, CH)])
            return 0
        lax.fori_loop(0, STRIPE // CH, _zpub, 0)
        plsc.subcore_barrier()

        _fill_idx(idxva, 0, ioff)
        pltpu.async_copy(h2_hbm.at[idxva], hbufa, sema)

        def _pair(jo, carry):
            a = 2 * jo
            b = a + 1

            @pl.when(b < NCH)
            def _():
                _fill_idx(idxvb, b, ioff)
                pltpu.async_copy(h2_hbm.at[idxvb], hbufb, semb)

            pltpu.make_async_copy(h2_hbm.at[idxva], hbufa, sema).wait()
            carry = _process(a, hbufa, carry)

            @pl.when(b + 1 < NCH)
            def _():
                _fill_idx(idxva, b + 1, ioff)
                pltpu.async_copy(h2_hbm.at[idxva], hbufa, sema)

            def _do_b(carry):
                pltpu.make_async_copy(h2_hbm.at[idxvb], hbufb, semb).wait()
                return _process(b, hbufb, carry)

            carry = lax.cond(b < NCH, _do_b, lambda cr: cr, carry)
            return carry

        s1v, s2v = lax.fori_loop(0, (NCH + 1) // 2, _pair,
                                 (zero16, zero16))
        plsc.subcore_barrier()

        def _expo(k, _):
            off = s * STRIPE + k * CH
            pltpu.sync_copy(o1sh.at[pl.ds(off, CH)], zo1)
            pltpu.sync_copy(zo1, o1_hbm.at[c, i, pl.ds(off, CH)])
            pltpu.sync_copy(gsh.at[pl.ds(off, CH)], zg)
            pltpu.sync_copy(zg, g_hbm.at[c, i, pl.ds(off, CH)])
            return 0
        lax.fori_loop(0, STRIPE // CH, _expo, 0)
        sbuf[pl.ds(0, 16)] = s1v
        pltpu.sync_copy(sbuf, s1_hbm.at[c, i, s])
        sbuf[pl.ds(0, 16)] = s2v
        pltpu.sync_copy(sbuf, s2_hbm.at[c, i, s])
        plsc.subcore_barrier()
        return 0

    lax.fori_loop(0, H, _head, 0)


def _sc_pass2(p, rden, srcr, dstr, h2, edge_fts):
    mesh = plsc.VectorSubcoreMesh(core_axis_name="c", subcore_axis_name="s")
    return pl.kernel(
        _sc_pass2_body,
        out_type=[
            jax.ShapeDtypeStruct((NC, H, NP_, DO), jnp.float32),
            jax.ShapeDtypeStruct((NC, H, NP_, DE), jnp.float32),
            jax.ShapeDtypeStruct((NC, H, NS, 16), jnp.float32),
            jax.ShapeDtypeStruct((NC, H, NS, 16), jnp.float32),
        ],
        mesh=mesh,
        compiler_params=pltpu.CompilerParams(
            needs_layout_passes=False, use_tc_tiling_on_sc=False),
        scratch_types=[
            pltpu.VMEM((NCH, CH), jnp.int32),
            pltpu.VMEM((NCH, CH), jnp.int32),
            pltpu.VMEM((NCH, CH), jnp.float32),
            pltpu.VMEM((NP_,), jnp.float32),
            pltpu.VMEM((CH,), jnp.int32),
            pltpu.VMEM((CH,), jnp.int32),
            pltpu.VMEM((CH, DO), jnp.float32),
            pltpu.VMEM((CH, DO), jnp.float32),
            pltpu.VMEM((CH, DE), jnp.float32),
            pltpu.VMEM_SHARED((NP_, DO), jnp.float32),
            pltpu.VMEM_SHARED((NP_, DE), jnp.float32),
            pltpu.VMEM((CH, DO), jnp.float32),
            pltpu.VMEM((CH, DE), jnp.float32),
            pltpu.VMEM((16,), jnp.float32),
            pltpu.SemaphoreType.DMA,
            pltpu.SemaphoreType.DMA,
        ],
    )(p, rden, srcr, dstr, h2, edge_fts)


def _final_body(o0_ref, o1_ref, g0_ref, g1_ref, we_ref, s1_ref, s2_ref,
                out_ref):
    s1 = jnp.sum(s1_ref[...], axis=(0, 2, 3))
    s2 = jnp.sum(s2_ref[...], axis=(0, 2, 3))
    fe = jnp.float32(E)
    var = s2 / fe - (s1 / fe) ** 2
    w = jnp.exp(var)
    w = w / jnp.sum(w)
    for i in range(H):
        g = g0_ref[i] + g1_ref[i]
        acc = o0_ref[i] + o1_ref[i] + jnp.dot(
            g, we_ref[i], preferred_element_type=jnp.float32)
        out_ref[:, i * DO:(i + 1) * DO] = acc * w[i]


def _final(o0, o1, g0, g1, We, s1, s2):
    nblk = N // BND
    return pl.pallas_call(
        _final_body,
        grid=(nblk,),
        in_specs=[
            pl.BlockSpec((H, BND, DO), lambda nb: (0, nb, 0)),
            pl.BlockSpec((H, BND, DO), lambda nb: (0, nb, 0)),
            pl.BlockSpec((H, BND, DE), lambda nb: (0, nb, 0)),
            pl.BlockSpec((H, BND, DE), lambda nb: (0, nb, 0)),
            pl.BlockSpec((H, DE, DO), lambda nb: (0, 0, 0)),
            pl.BlockSpec((NC, H, NS, 16), lambda nb: (0, 0, 0, 0)),
            pl.BlockSpec((NC, H, NS, 16), lambda nb: (0, 0, 0, 0)),
        ],
        out_specs=pl.BlockSpec((BND, H * DO), lambda nb: (nb, 0)),
        out_shape=jax.ShapeDtypeStruct((N, H * DO), jnp.float32),
    )(o0, o1, g0, g1, We, s1, s2)


@jax.jit
def kernel(node_fts, edge_fts, edges, Wh, We, a_src, a_dst, a_edge):
    node_p = jnp.pad(node_fts, ((0, NP_ - N), (0, 0)))
    edges32 = edges.astype(jnp.int32)
    srcr = edges32[:, 0].reshape(NW, NCH, CH)
    dstr = edges32[:, 1].reshape(NW, NCH, CH)

    h = _prep_nodes(node_p, Wh)
    s8, smax8 = _prep_s(node_p.T, Wh.transpose(0, 2, 1), a_src, a_dst)
    t, tmax = _prep_edges(edge_fts.T, We.transpose(0, 2, 1), a_edge)
    t_r = t.reshape(H, NW, NCH, CH)

    p, den = _sc_pass1(s8, t_r, srcr, dstr, smax8, tmax)
    rden = _rdenom(den)

    h2 = h.reshape(H * NP_, DO)
    o1p, gp, s1, s2 = _sc_pass2(p, rden, srcr, dstr, h2, edge_fts)

    return _final(o1p[0], o1p[1], gp[0], gp[1], We, s1, s2)


# speedup vs baseline: 25.2368x; 1.0012x over previous
import functools
import jax
import jax.numpy as jnp
from jax import lax
from jax.experimental import pallas as pl
from jax.experimental.pallas import tpu as pltpu
from jax.experimental.pallas import tpu_sc as plsc

N = 10000
E = 320000
DIN = 128
DE = 16
DO = 64
H = 4
ALPHA = 0.2

NP_ = 10240
NC = 2
NS = 16
NW = NC * NS
EW = E // NW
CH = 80
NCH = EW // CH
STRIPE = NP_ // NS

BN = 2048
BE = 12800
BND = 2000


def _prep_nodes_body(node_ref, wh_ref, h_ref):
    x = node_ref[...]
    h_ref[0] = jnp.dot(x, wh_ref[0], preferred_element_type=jnp.float32)


def _prep_nodes(node_p, Wh):
    nblk = NP_ // BN
    return pl.pallas_call(
        _prep_nodes_body,
        grid=(H, nblk),
        in_specs=[
            pl.BlockSpec((BN, DIN), lambda i, nb: (nb, 0)),
            pl.BlockSpec((1, DIN, DO), lambda i, nb: (i, 0, 0)),
        ],
        out_specs=pl.BlockSpec((1, BN, DO), lambda i, nb: (i, nb, 0)),
        out_shape=jax.ShapeDtypeStruct((H, NP_, DO), jnp.float32),
    )(node_p, Wh)


def _prep_s_body(xt_ref, wht_ref, asrc_ref, adst_ref, s8_ref, smax_ref):
    nb = pl.program_id(0)
    rows = [jnp.dot(asrc_ref[i], wht_ref[i], preferred_element_type=jnp.float32)
            for i in range(H)]
    rows += [jnp.dot(adst_ref[i], wht_ref[i], preferred_element_type=jnp.float32)
             for i in range(H)]
    wst = jnp.concatenate(rows, axis=0)
    sblk = jnp.dot(wst, xt_ref[...], preferred_element_type=jnp.float32)
    s8_ref[...] = sblk
    mx = jnp.max(sblk, axis=1)
    bc = jnp.broadcast_to(mx[:, None], (2 * H, 16))

    @pl.when(nb == 0)
    def _():
        smax_ref[:, 0, :] = bc

    @pl.when(nb != 0)
    def _():
        smax_ref[:, 0, :] = jnp.maximum(smax_ref[:, 0, :], bc)


def _prep_s(node_pT, WhT, a_src, a_dst):
    nblk = NP_ // BN
    return pl.pallas_call(
        _prep_s_body,
        grid=(nblk,),
        in_specs=[
            pl.BlockSpec((DIN, BN), lambda nb: (0, nb)),
            pl.BlockSpec((H, DO, DIN), lambda nb: (0, 0, 0)),
            pl.BlockSpec((H, 1, DO), lambda nb: (0, 0, 0)),
            pl.BlockSpec((H, 1, DO), lambda nb: (0, 0, 0)),
        ],
        out_specs=[
            pl.BlockSpec((2 * H, BN), lambda nb: (0, nb)),
            pl.BlockSpec((2 * H, 1, 16), lambda nb: (0, 0, 0)),
        ],
        out_shape=[
            jax.ShapeDtypeStruct((2 * H, NP_), jnp.float32),
            jax.ShapeDtypeStruct((2 * H, 1, 16), jnp.float32),
        ],
    )(node_pT, WhT, a_src.reshape(H, 1, DO), a_dst.reshape(H, 1, DO))


def _prep_edges_body(eft_ref, wet_ref, ae_ref, t_ref, tmax_ref):
    eb = pl.program_id(0)
    rows = [jnp.dot(ae_ref[i], wet_ref[i], preferred_element_type=jnp.float32)
            for i in range(H)]
    v4 = jnp.concatenate(rows, axis=0)
    tblk = jnp.dot(v4, eft_ref[...], preferred_element_type=jnp.float32)
    t_ref[...] = tblk
    mx = jnp.max(tblk, axis=1)
    bc = jnp.broadcast_to(mx[:, None], (H, 16))

    @pl.when(eb == 0)
    def _():
        tmax_ref[:, 0, :] = bc

    @pl.when(eb != 0)
    def _():
        tmax_ref[:, 0, :] = jnp.maximum(tmax_ref[:, 0, :], bc)


def _prep_edges(edge_ftsT, WeT, a_edge):
    return pl.pallas_call(
        _prep_edges_body,
        grid=(E // BE,),
        in_specs=[
            pl.BlockSpec((DE, BE), lambda eb: (0, eb)),
            pl.BlockSpec((H, DO, DE), lambda eb: (0, 0, 0)),
            pl.BlockSpec((H, 1, DO), lambda eb: (0, 0, 0)),
        ],
        out_specs=[
            pl.BlockSpec((H, BE), lambda eb: (0, eb)),
            pl.BlockSpec((H, 1, 16), lambda eb: (0, 0, 0)),
        ],
        out_shape=[
            jax.ShapeDtypeStruct((H, E), jnp.float32),
            jax.ShapeDtypeStruct((H, 1, 16), jnp.float32),
        ],
    )(edge_ftsT, WeT, a_edge.reshape(H, 1, DO))


def _sc_pass1_body(s8_hbm, t_hbm, srcr_hbm, dstr_hbm,
                   smax_hbm, tmax_hbm,
                   p_hbm, den_hbm,
                   src2d, dst2d, t2d, p2d, stab, dtab, mv1, mv2,
                   dpriv, dsh, dbuf, abuf):
    c = lax.axis_index("c")
    s = lax.axis_index("s")
    wid = c * NS + s
    pltpu.sync_copy(srcr_hbm.at[wid], src2d)
    pltpu.sync_copy(dstr_hbm.at[wid], dst2d)
    zero16 = jnp.zeros((16,), jnp.float32)
    for i in range(H):
        pltpu.sync_copy(s8_hbm.at[i], stab)
        pltpu.sync_copy(s8_hbm.at[H + i], dtab)
        pltpu.sync_copy(t_hbm.at[i, wid], t2d)
        pltpu.sync_copy(smax_hbm.at[i, 0], mv1)
        pltpu.sync_copy(tmax_hbm.at[i, 0], mv2)
        mvv = mv1[...] + mv2[...]

        def _zpriv(k, _):
            dpriv[pl.ds(k * 16, 16)] = zero16
            return 0
        lax.fori_loop(0, NP_ // 16, _zpriv, 0)

        def _chunk(j, _):
            for q in range(CH // 16):
                sl = pl.ds(q * 16, 16)
                si = src2d[j, sl]
                di = dst2d[j, sl]
                a = plsc.load_gather(stab, [si])
                b = plsc.load_gather(dtab, [di])
                e = a + b + t2d[j, sl]
                z = jnp.where(e >= 0, e, e * ALPHA)
                u = b + mvv
                cv = jnp.where(u >= 0, u, u * ALPHA)
                p16 = jnp.exp(z - cv)
                p2d[j, sl] = p16
                plsc.addupdate_scatter(dpriv, [di], p16)
            return 0
        lax.fori_loop(0, NCH, _chunk, 0)

        pltpu.sync_copy(p2d, p_hbm.at[i, wid])
        pltpu.sync_copy(dpriv, dsh.at[s])
        plsc.subcore_barrier()

        def _zab(k, _):
            abuf[pl.ds(k * 16, 16)] = zero16
            return 0
        lax.fori_loop(0, STRIPE // 16, _zab, 0)
        for m in range(NS):
            pltpu.sync_copy(dsh.at[m, pl.ds(s * STRIPE, STRIPE)], dbuf)

            def _acc(k, _):
                sl = pl.ds(k * 16, 16)
                abuf[sl] = abuf[sl] + dbuf[sl]
                return 0
            lax.fori_loop(0, STRIPE // 16, _acc, 0)
        pltpu.sync_copy(abuf, den_hbm.at[c, i, pl.ds(s * STRIPE, STRIPE)])
        plsc.subcore_barrier()


def _sc_pass1(s8, t, srcr, dstr, smax, tmax):
    mesh = plsc.VectorSubcoreMesh(core_axis_name="c", subcore_axis_name="s")
    return pl.kernel(
        _sc_pass1_body,
        out_type=[
            jax.ShapeDtypeStruct((H, NW, NCH, CH), jnp.float32),
            jax.ShapeDtypeStruct((NC, H, NP_), jnp.float32),
        ],
        mesh=mesh,
        compiler_params=pltpu.CompilerParams(needs_layout_passes=False),
        scratch_types=[
            pltpu.VMEM((NCH, CH), jnp.int32),
            pltpu.VMEM((NCH, CH), jnp.int32),
            pltpu.VMEM((NCH, CH), jnp.float32),
            pltpu.VMEM((NCH, CH), jnp.float32),
            pltpu.VMEM((NP_,), jnp.float32),
            pltpu.VMEM((NP_,), jnp.float32),
            pltpu.VMEM((16,), jnp.float32),
            pltpu.VMEM((16,), jnp.float32),
            pltpu.VMEM((NP_,), jnp.float32),
            pltpu.VMEM_SHARED((NS, NP_), jnp.float32),
            pltpu.VMEM((STRIPE,), jnp.float32),
            pltpu.VMEM((STRIPE,), jnp.float32),
        ],
    )(s8, t, srcr, dstr, smax, tmax)


def _rdenom_body(den_ref, out_ref):
    d = den_ref[0] + den_ref[1]
    out_ref[...] = 1.0 / (d + 1e-16)


def _rdenom(den):
    return pl.pallas_call(
        _rdenom_body,
        grid=(H,),
        in_specs=[pl.BlockSpec((NC, NP_), lambda i: (0, i))],
        out_specs=pl.BlockSpec((NP_,), lambda i: (i,)),
        out_shape=jax.ShapeDtypeStruct((H * NP_,), jnp.float32),
    )(den.reshape(NC, H * NP_))


def _sc_pass2_body(p_hbm, rden_hbm, srcr_hbm, dstr_hbm, h2_hbm, ef_hbm,
                   o1_hbm, g_hbm, s1_hbm, s2_hbm,
                   src2d, dst2d, p2d, rtab, idxva, idxvb, hbufa, hbufb,
                   efbufa, efbufb, o1sh, gsh, zo1, zg, sbuf,
                   sema, semb, semoa, semga, semob, semgb):
    c = lax.axis_index("c")
    s = lax.axis_index("s")
    wid = c * NS + s
    base = wid * EW
    pltpu.sync_copy(srcr_hbm.at[wid], src2d)
    pltpu.sync_copy(dstr_hbm.at[wid], dst2d)
    zero16 = jnp.zeros((16,), jnp.float32)

    def _fill_idx(idxv, j, ioff):
        for q in range(CH // 16):
            sl = pl.ds(q * 16, 16)
            idxv[sl] = src2d[j, sl] + ioff

    def _process(j, hbuf, efbuf, semo, semg, carry):
        s1v, s2v = carry
        pltpu.sync_copy(ef_hbm.at[pl.ds(base + j * CH, CH)], efbuf)
        for q in range(CH // 16):
            sl = pl.ds(q * 16, 16)
            di = dst2d[j, sl]
            r16 = plsc.load_gather(rtab, [di])
            a16 = p2d[j, sl] * r16
            s1v = s1v + a16
            s2v = s2v + a16 * a16
            for r in range(16):
                rr = q * 16 + r
                av = a16[r]
                for m in range(DO // 16):
                    sl2 = pl.ds(m * 16, 16)
                    hbuf[rr, sl2] = hbuf[rr, sl2] * av
                efbuf[rr, pl.ds(0, 16)] = efbuf[rr, pl.ds(0, 16)] * av
        pltpu.async_copy(hbuf, o1sh.at[dst2d.at[j]], semo, add=True)
        pltpu.async_copy(efbuf, gsh.at[dst2d.at[j]], semg, add=True)
        return (s1v, s2v)

    def _wait_scatters(hbuf, efbuf, semo, semg):
        pltpu.make_async_copy(hbuf, o1sh.at[dst2d.at[0]], semo).wait()
        pltpu.make_async_copy(efbuf, gsh.at[dst2d.at[0]], semg).wait()

    def _head(i, _):
        ioff = i * NP_
        pltpu.sync_copy(p_hbm.at[i, wid], p2d)
        pltpu.sync_copy(rden_hbm.at[pl.ds(i * NP_, NP_)], rtab)

        def _zrow(k, _):
            for m in range(DO // 16):
                zo1[k, pl.ds(m * 16, 16)] = zero16
            zg[k, pl.ds(0, 16)] = zero16
            return 0
        lax.fori_loop(0, CH, _zrow, 0)

        def _zpub(k, _):
            pltpu.sync_copy(zo1, o1sh.at[pl.ds(s * STRIPE + k * CH, CH)])
            pltpu.sync_copy(zg, gsh.at[pl.ds(s * STRIPE + k * CH, CH)])
            return 0
        lax.fori_loop(0, STRIPE // CH, _zpub, 0)
        plsc.subcore_barrier()

        _fill_idx(idxva, 0, ioff)
        pltpu.async_copy(h2_hbm.at[idxva], hbufa, sema)

        def _pair(jo, carry):
            a = 2 * jo
            b = a + 1

            @pl.when(b < NCH)
            def _():
                @pl.when(jo > 0)
                def _():
                    _wait_scatters(hbufb, efbufb, semob, semgb)
                _fill_idx(idxvb, b, ioff)
                pltpu.async_copy(h2_hbm.at[idxvb], hbufb, semb)

            pltpu.make_async_copy(h2_hbm.at[idxva], hbufa, sema).wait()
            carry = _process(a, hbufa, efbufa, semoa, semga, carry)

            def _do_b(carry):
                pltpu.make_async_copy(h2_hbm.at[idxvb], hbufb, semb).wait()
                return _process(b, hbufb, efbufb, semob, semgb, carry)

            carry = lax.cond(b < NCH, _do_b, lambda cr: cr, carry)

            @pl.when(b + 1 < NCH)
            def _():
                _wait_scatters(hbufa, efbufa, semoa, semga)
                _fill_idx(idxva, b + 1, ioff)
                pltpu.async_copy(h2_hbm.at[idxva], hbufa, sema)

            return carry

        s1v, s2v = lax.fori_loop(0, (NCH + 1) // 2, _pair,
                                 (zero16, zero16))
        _wait_scatters(hbufa, efbufa, semoa, semga)
        _wait_scatters(hbufb, efbufb, semob, semgb)
        plsc.subcore_barrier()

        def _expo(k, _):
            off = s * STRIPE + k * CH
            pltpu.sync_copy(o1sh.at[pl.ds(off, CH)], zo1)
            pltpu.sync_copy(zo1, o1_hbm.at[c, i, pl.ds(off, CH)])
            pltpu.sync_copy(gsh.at[pl.ds(off, CH)], zg)
            pltpu.sync_copy(zg, g_hbm.at[c, i, pl.ds(off, CH)])
            return 0
        lax.fori_loop(0, STRIPE // CH, _expo, 0)
        sbuf[pl.ds(0, 16)] = s1v
        pltpu.sync_copy(sbuf, s1_hbm.at[c, i, s])
        sbuf[pl.ds(0, 16)] = s2v
        pltpu.sync_copy(sbuf, s2_hbm.at[c, i, s])
        plsc.subcore_barrier()
        return 0

    lax.fori_loop(0, H, _head, 0)


def _sc_pass2(p, rden, srcr, dstr, h2, edge_fts):
    mesh = plsc.VectorSubcoreMesh(core_axis_name="c", subcore_axis_name="s")
    return pl.kernel(
        _sc_pass2_body,
        out_type=[
            jax.ShapeDtypeStruct((NC, H, NP_, DO), jnp.float32),
            jax.ShapeDtypeStruct((NC, H, NP_, DE), jnp.float32),
            jax.ShapeDtypeStruct((NC, H, NS, 16), jnp.float32),
            jax.ShapeDtypeStruct((NC, H, NS, 16), jnp.float32),
        ],
        mesh=mesh,
        compiler_params=pltpu.CompilerParams(
            needs_layout_passes=False, use_tc_tiling_on_sc=False),
        scratch_types=[
            pltpu.VMEM((NCH, CH), jnp.int32),
            pltpu.VMEM((NCH, CH), jnp.int32),
            pltpu.VMEM((NCH, CH), jnp.float32),
            pltpu.VMEM((NP_,), jnp.float32),
            pltpu.VMEM((CH,), jnp.int32),
            pltpu.VMEM((CH,), jnp.int32),
            pltpu.VMEM((CH, DO), jnp.float32),
            pltpu.VMEM((CH, DO), jnp.float32),
            pltpu.VMEM((CH, DE), jnp.float32),
            pltpu.VMEM((CH, DE), jnp.float32),
            pltpu.VMEM_SHARED((NP_, DO), jnp.float32),
            pltpu.VMEM_SHARED((NP_, DE), jnp.float32),
            pltpu.VMEM((CH, DO), jnp.float32),
            pltpu.VMEM((CH, DE), jnp.float32),
            pltpu.VMEM((16,), jnp.float32),
            pltpu.SemaphoreType.DMA,
            pltpu.SemaphoreType.DMA,
            pltpu.SemaphoreType.DMA,
            pltpu.SemaphoreType.DMA,
            pltpu.SemaphoreType.DMA,
            pltpu.SemaphoreType.DMA,
        ],
    )(p, rden, srcr, dstr, h2, edge_fts)


def _final_body(o0_ref, o1_ref, g0_ref, g1_ref, we_ref, s1_ref, s2_ref,
                out_ref):
    s1 = jnp.sum(s1_ref[...], axis=(0, 2, 3))
    s2 = jnp.sum(s2_ref[...], axis=(0, 2, 3))
    fe = jnp.float32(E)
    var = s2 / fe - (s1 / fe) ** 2
    w = jnp.exp(var)
    w = w / jnp.sum(w)
    for i in range(H):
        g = g0_ref[i] + g1_ref[i]
        acc = o0_ref[i] + o1_ref[i] + jnp.dot(
            g, we_ref[i], preferred_element_type=jnp.float32)
        out_ref[:, i * DO:(i + 1) * DO] = acc * w[i]


def _final(o0, o1, g0, g1, We, s1, s2):
    nblk = N // BND
    return pl.pallas_call(
        _final_body,
        grid=(nblk,),
        in_specs=[
            pl.BlockSpec((H, BND, DO), lambda nb: (0, nb, 0)),
            pl.BlockSpec((H, BND, DO), lambda nb: (0, nb, 0)),
            pl.BlockSpec((H, BND, DE), lambda nb: (0, nb, 0)),
            pl.BlockSpec((H, BND, DE), lambda nb: (0, nb, 0)),
            pl.BlockSpec((H, DE, DO), lambda nb: (0, 0, 0)),
            pl.BlockSpec((NC, H, NS, 16), lambda nb: (0, 0, 0, 0)),
            pl.BlockSpec((NC, H, NS, 16), lambda nb: (0, 0, 0, 0)),
        ],
        out_specs=pl.BlockSpec((BND, H * DO), lambda nb: (nb, 0)),
        out_shape=jax.ShapeDtypeStruct((N, H * DO), jnp.float32),
    )(o0, o1, g0, g1, We, s1, s2)


@jax.jit
def kernel(node_fts, edge_fts, edges, Wh, We, a_src, a_dst, a_edge):
    node_p = jnp.pad(node_fts, ((0, NP_ - N), (0, 0)))
    edges32 = edges.astype(jnp.int32)
    srcr = edges32[:, 0].reshape(NW, NCH, CH)
    dstr = edges32[:, 1].reshape(NW, NCH, CH)

    h = _prep_nodes(node_p, Wh)
    s8, smax8 = _prep_s(node_p.T, Wh.transpose(0, 2, 1), a_src, a_dst)
    t, tmax = _prep_edges(edge_fts.T, We.transpose(0, 2, 1), a_edge)
    t_r = t.reshape(H, NW, NCH, CH)

    p, den = _sc_pass1(s8, t_r, srcr, dstr, smax8, tmax)
    rden = _rdenom(den)

    h2 = h.reshape(H * NP_, DO)
    o1p, gp, s1, s2 = _sc_pass2(p, rden, srcr, dstr, h2, edge_fts)

    return _final(o1p[0], o1p[1], gp[0], gp[1], We, s1, s2)
